# baseline (speedup 1.0000x reference)
import functools
import math

import jax
import jax.numpy as jnp
from jax import lax
from jax.experimental import pallas as pl
from jax.experimental.pallas import tpu as pltpu
from jax.experimental.pallas import tpu_sc as plsc

D_MODEL = 64
SCALE_F = math.sqrt(D_MODEL)
CHUNK = 128
SPS = 2
LANES = 16
NW = 32
N_S = 200
N_B = 4096


@jax.jit
def _embed_lookup(x4, embedding):
    n_steps = N_S // SPS
    mesh = plsc.VectorSubcoreMesh(core_axis_name="c", subcore_axis_name="s")
    num_cores = 2

    @functools.partial(
        pl.kernel,
        mesh=mesh,
        out_type=jax.ShapeDtypeStruct((N_S, 8, NW, 8, CHUNK), jnp.float32),
        compiler_params=pltpu.CompilerParams(
            use_tc_tiling_on_sc=False, needs_layout_passes=False
        ),
        scratch_types=[
            pltpu.VMEM((N_S // 8, 1, 8, CHUNK), jnp.int32),
            pltpu.VMEM((SPS * CHUNK, D_MODEL), jnp.float32),
            pltpu.VMEM((SPS * CHUNK, D_MODEL), jnp.float32),
            pltpu.VMEM((SPS, 8, 1, 8, CHUNK + 1), jnp.float32),
            pltpu.VMEM((SPS, 8, 1, 8, CHUNK), jnp.float32),
            pltpu.VMEM((SPS, 8, 1, 8, CHUNK), jnp.float32),
            pltpu.SemaphoreType.DMA,
            pltpu.SemaphoreType.DMA,
            pltpu.SemaphoreType.DMA,
            pltpu.SemaphoreType.DMA,
        ],
    )
    def k(x_hbm, table_hbm, out_hbm, idx_v, ga, gb, ost, oa, ob,
          ga_s, gb_s, oa_s, ob_s):
        wid = lax.axis_index("s") * num_cores + lax.axis_index("c")
        gbuf = (ga, gb)
        obuf = (oa, ob)
        gsem = (ga_s, gb_s)
        osem = (oa_s, ob_s)

        pltpu.sync_copy(x_hbm.at[:, pl.ds(wid, 1)], idx_v)

        dts = [jnp.arange(16, dtype=jnp.int32) // 8 + (c * 2)
               for c in range(D_MODEL // LANES)]
        drs = [jnp.arange(16, dtype=jnp.int32) % 8
               for _ in range(D_MODEL // LANES)]
        zero16 = jnp.zeros((16,), dtype=jnp.int32)

        def start_gather(j, b):
            for sl in range(SPS):
                s = j * SPS + sl
                pltpu.async_copy(
                    table_hbm.at[idx_v.at[s // 8, 0, s % 8]],
                    gbuf[b].at[pl.ds(sl * CHUNK, CHUNK)],
                    gsem[b],
                )

        def drain_gather(b):
            pltpu.make_async_copy(
                table_hbm.at[pl.ds(0, SPS * CHUNK)], gbuf[b], gsem[b]
            ).wait()

        def out_copy(j, b):
            return pltpu.make_async_copy(
                obuf[b],
                out_hbm.at[pl.ds(j * SPS, SPS), slice(None), pl.ds(wid, 1)],
                osem[b],
            )

        start_gather(0, 0)

        def outer(i, carry):
            j0 = i * 2
            for b in range(2):
                nb = 1 - b
                j = j0 + b

                drain_gather(b)

                @pl.when(j + 1 < n_steps)
                def _():
                    start_gather(j + 1, nb)

                @pl.when(j >= 2)
                def _():
                    out_copy(j - 2, b).wait()

                for sl in range(SPS):
                    sl_v = jnp.full((16,), sl, dtype=jnp.int32)

                    def transform(bl, c2, sl=sl, sl_v=sl_v):
                        bl_v = jnp.full((16,), bl, dtype=jnp.int32)
                        for c in range(D_MODEL // LANES):
                            v = gbuf[b][sl * CHUNK + bl, pl.ds(c * LANES, LANES)]
                            plsc.store_scatter(
                                ost, [sl_v, dts[c], zero16, drs[c], bl_v],
                                v * SCALE_F,
                            )
                        return c2

                    lax.fori_loop(0, CHUNK, transform, 0, unroll=4)

                    def compact(q, c2, sl=sl):
                        dt2 = q // 8
                        dr2 = q % 8
                        for kk in range(CHUNK // LANES):
                            csl = pl.ds(kk * LANES, LANES)
                            obuf[b][sl, dt2, 0, dr2, csl] = ost[sl, dt2, 0, dr2, csl]
                        return c2

                    lax.fori_loop(0, 64, compact, 0, unroll=2)

                out_copy(j, b).start()
            return carry

        lax.fori_loop(0, n_steps // 2, outer, 0)
        out_copy(n_steps - 2, 0).wait()
        out_copy(n_steps - 1, 1).wait()

    return k(x4, embedding)


def kernel(x, embedding):
    x4 = x.T.reshape(N_S // 8, 8, NW, CHUNK).transpose(0, 2, 1, 3)
    p6 = _embed_lookup(x4, embedding)
    return p6.transpose(2, 4, 0, 1, 3).reshape(N_B, N_S, D_MODEL)


# speedup vs baseline: 1.2783x; 1.2783x over previous
import functools
import math

import jax
import jax.numpy as jnp
from jax import lax
from jax.experimental import pallas as pl
from jax.experimental.pallas import tpu as pltpu
from jax.experimental.pallas import tpu_sc as plsc

D_MODEL = 64
SCALE_F = math.sqrt(D_MODEL)
CHUNK = 128
SPS = 2
LANES = 16
NW = 32
N_S = 200
N_B = 4096


@jax.jit
def _embed_lookup(x4, embedding):
    n_steps = N_S // SPS
    mesh = plsc.VectorSubcoreMesh(core_axis_name="c", subcore_axis_name="s")
    num_cores = 2

    @functools.partial(
        pl.kernel,
        mesh=mesh,
        out_type=jax.ShapeDtypeStruct((N_S, 8, NW, 8, CHUNK), jnp.float32),
        compiler_params=pltpu.CompilerParams(
            use_tc_tiling_on_sc=False, needs_layout_passes=False
        ),
        scratch_types=[
            pltpu.VMEM((N_S // 8, 1, 8, CHUNK), jnp.int32),
            pltpu.VMEM((SPS * CHUNK, D_MODEL), jnp.float32),
            pltpu.VMEM((SPS * CHUNK, D_MODEL), jnp.float32),
            pltpu.VMEM((SPS, 8, 1, 8, CHUNK + 1), jnp.float32),
            pltpu.VMEM((SPS, 8, 1, 8, CHUNK + 1), jnp.float32),
            pltpu.SemaphoreType.DMA,
            pltpu.SemaphoreType.DMA,
            pltpu.SemaphoreType.DMA,
            pltpu.SemaphoreType.DMA,
        ],
    )
    def k(x_hbm, table_hbm, out_hbm, idx_v, ga, gb, oa, ob, ga_s, gb_s, oa_s, ob_s):
        wid = lax.axis_index("s") * num_cores + lax.axis_index("c")
        gbuf = (ga, gb)
        obuf = (oa, ob)
        gsem = (ga_s, gb_s)
        osem = (oa_s, ob_s)

        pltpu.sync_copy(x_hbm.at[:, pl.ds(wid, 1)], idx_v)

        dts = [jnp.arange(16, dtype=jnp.int32) // 8 + (c * 2)
               for c in range(D_MODEL // LANES)]
        drs = [jnp.arange(16, dtype=jnp.int32) % 8
               for _ in range(D_MODEL // LANES)]
        zero16 = jnp.zeros((16,), dtype=jnp.int32)

        def start_gather(j, b):
            for sl in range(SPS):
                s = j * SPS + sl
                pltpu.async_copy(
                    table_hbm.at[idx_v.at[s // 8, 0, s % 8]],
                    gbuf[b].at[pl.ds(sl * CHUNK, CHUNK)],
                    gsem[b],
                )

        def drain_gather(b):
            pltpu.make_async_copy(
                table_hbm.at[pl.ds(0, SPS * CHUNK)], gbuf[b], gsem[b]
            ).wait()

        def out_copy(j, b):
            return pltpu.make_async_copy(
                obuf[b].at[:, :, :, :, pl.ds(0, CHUNK)],
                out_hbm.at[pl.ds(j * SPS, SPS), slice(None), pl.ds(wid, 1)],
                osem[b],
            )

        start_gather(0, 0)

        def outer(i, carry):
            j0 = i * 2
            for b in range(2):
                nb = 1 - b
                j = j0 + b

                drain_gather(b)

                @pl.when(j + 1 < n_steps)
                def _():
                    start_gather(j + 1, nb)

                @pl.when(j >= 2)
                def _():
                    out_copy(j - 2, b).wait()

                for sl in range(SPS):
                    sl_v = jnp.full((16,), sl, dtype=jnp.int32)

                    def transform(bl, c2, sl=sl, sl_v=sl_v):
                        bl_v = jnp.full((16,), bl, dtype=jnp.int32)
                        for c in range(D_MODEL // LANES):
                            v = gbuf[b][sl * CHUNK + bl, pl.ds(c * LANES, LANES)]
                            plsc.store_scatter(
                                obuf[b], [sl_v, dts[c], zero16, drs[c], bl_v],
                                v * SCALE_F,
                            )
                        return c2

                    lax.fori_loop(0, CHUNK, transform, 0, unroll=4)

                out_copy(j, b).start()
            return carry

        lax.fori_loop(0, n_steps // 2, outer, 0)
        out_copy(n_steps - 2, 0).wait()
        out_copy(n_steps - 1, 1).wait()

    return k(x4, embedding)


def kernel(x, embedding):
    x4 = x.T.reshape(N_S // 8, 8, NW, CHUNK).transpose(0, 2, 1, 3)
    p6 = _embed_lookup(x4, embedding)
    return p6.transpose(2, 4, 0, 1, 3).reshape(N_B, N_S, D_MODEL)
